# Initial kernel scaffold; baseline (speedup 1.0000x reference)
import jax
import jax.numpy as jnp
from jax.experimental import pallas as pl


def kernel(heatmap, offset, wh, landmark):
    raise NotImplementedError("write your pallas kernel here")


# speedup vs baseline: 2.8346x; 2.8346x over previous
import functools

import jax
import jax.numpy as jnp
from jax import lax
from jax.experimental import pallas as pl
from jax.experimental.pallas import tpu as pltpu
from jax.experimental.pallas import tpu_sc as plsc

TOPK = 100
NSEL = 112
NB = 512
CAP = 512
IMG = 128
HW = IMG * IMG
BATCH = 16
SCALE = 4.0
THRESH = 0.01
PW = 144


def _sc_body(hm, off, wh, lm, o_ids, o_sc, o_bb, o_lm,
             p_ref, hx_ref, m_ref, hist_ref, tot_ref, cum_ref,
             cval_ref, cidx_ref, sval_ref, sidx_ref,
             gidx_ref, gbuf_ref, ob_ids, ob_sc, ob_bb, ob_lm, sem):
  wid = lax.axis_index("s") * 2 + lax.axis_index("c")

  @pl.when(wid < BATCH)
  def _():
    b = wid
    lane = lax.iota(jnp.int32, 16)
    neg1f = jnp.full((16,), -1.0, jnp.float32)
    one_i = jnp.full((16,), 1, jnp.int32)
    zero_i = jnp.full((16,), 0, jnp.int32)

    pltpu.sync_copy(hm.at[pl.ds(b * HW, HW)], m_ref)

    pad_col = jnp.where(lane == 0, 0, 129)
    pad_msk = lane < 2
    def fill_row(r, carry):
      for j in range(8):
        p_ref[pl.ds(r * PW + 1 + 16 * j, 16)] = m_ref[pl.ds(r * IMG + 16 * j, 16)]
      plsc.store_scatter(p_ref, [r * PW + pad_col], neg1f, mask=pad_msk)
      return carry
    lax.fori_loop(0, IMG, fill_row, 0)

    for j in range(8):
      hx_ref[0, pl.ds(16 * j, 16)] = neg1f
      hx_ref[129, pl.ds(16 * j, 16)] = neg1f

    def hrow(r, carry):
      for j in range(8):
        a = p_ref[pl.ds(r * PW + 16 * j, 16)]
        bb_ = p_ref[pl.ds(r * PW + 16 * j + 1, 16)]
        cc = p_ref[pl.ds(r * PW + 16 * j + 2, 16)]
        hx_ref[r + 1, pl.ds(16 * j, 16)] = jnp.maximum(jnp.maximum(a, bb_), cc)
      return carry
    lax.fori_loop(0, IMG, hrow, 0)

    def zh(i, carry):
      hist_ref[pl.ds(i * 16, 16)] = zero_i
      return carry
    lax.fori_loop(0, 16 * NB // 16, zh, 0)

    def prow(r, carry):
      for j in range(8):
        ctr = p_ref[pl.ds(r * PW + 16 * j + 1, 16)]
        v0 = hx_ref[r, pl.ds(16 * j, 16)]
        v1 = hx_ref[r + 1, pl.ds(16 * j, 16)]
        v2 = hx_ref[r + 2, pl.ds(16 * j, 16)]
        pool = jnp.maximum(jnp.maximum(v0, v1), v2)
        keep = ctr >= pool
        m_ref[pl.ds(r * IMG + 16 * j, 16)] = jnp.where(keep, ctr, -1.0)
        bidx = jnp.minimum((ctr * float(NB)).astype(jnp.int32), NB - 1)
        plsc.addupdate_scatter(hist_ref, [lane * NB + bidx], one_i, mask=keep)
      return carry
    lax.fori_loop(0, IMG, prow, 0)

    def tchunk(j, carry):
      acc = hist_ref[pl.ds(j * 16, 16)]
      for l in range(1, 16):
        acc = acc + hist_ref[pl.ds(l * NB + j * 16, 16)]
      tot_ref[pl.ds(j * 16, 16)] = acc
      return carry
    lax.fori_loop(0, NB // 16, tchunk, 0)

    def cchunk(j, carry):
      cs = jnp.cumsum(tot_ref[pl.ds(16 * j, 16)]) + carry
      cum_ref[pl.ds(16 * j, 16)] = cs
      return jnp.max(cs)
    tot_peaks = lax.fori_loop(0, NB // 16, cchunk, jnp.int32(0))

    def kchunk(j, kacc):
      tch = tot_ref[pl.ds(16 * j, 16)]
      cs = cum_ref[pl.ds(16 * j, 16)]
      sfx = tot_peaks - cs + tch
      return kacc + jnp.sum((sfx >= TOPK).astype(jnp.int32))
    t = lax.fori_loop(0, NB // 16, kchunk, jnp.int32(0)) - 1

    def pfc(i, carry):
      cval_ref[pl.ds(i * 16, 16)] = neg1f
      return carry
    lax.fori_loop(0, CAP // 16, pfc, 0)
    for j in range(NSEL // 16):
      sval_ref[pl.ds(16 * j, 16)] = neg1f
      sidx_ref[pl.ds(16 * j, 16)] = zero_i

    def comp(i, off_n):
      mv = m_ref[pl.ds(16 * i, 16)]
      keep = mv >= 0.0
      bidx = jnp.minimum((mv * float(NB)).astype(jnp.int32), NB - 1)
      sel = keep & (bidx >= t)
      cnt = jnp.cumsum(sel.astype(jnp.int32))
      dest = off_n + cnt - 1
      sel = sel & (dest < CAP)
      plsc.store_scatter(cval_ref, [dest], mv, mask=sel)
      plsc.store_scatter(cidx_ref, [dest], 16 * i + lane, mask=sel)
      return off_n + jnp.max(cnt)
    ncand = lax.fori_loop(0, HW // 16, comp, jnp.int32(0))
    nv = (jnp.minimum(ncand, CAP) + 15) // 16

    def select_k(k, carry):
      def scan_j(j, bc):
        bv, bp = bc
        v = cval_ref[pl.ds(16 * j, 16)]
        p = 16 * j + lane
        upd = v > bv
        return (jnp.where(upd, v, bv), jnp.where(upd, p, bp))
      bv, bp = lax.fori_loop(0, nv, scan_j,
                             (jnp.full((16,), -2.0, jnp.float32), zero_i))
      mx = jnp.max(bv)
      pmin = jnp.min(jnp.where(bv == mx, bp, CAP))
      pminv = zero_i + pmin
      kv = zero_i + k
      lane0 = lane == 0
      siv = plsc.load_gather(cidx_ref, [pminv])
      plsc.store_scatter(sval_ref, [kv], jnp.zeros((16,), jnp.float32) + mx,
                         mask=lane0)
      plsc.store_scatter(sidx_ref, [kv], siv, mask=lane0)
      plsc.store_scatter(cval_ref, [pminv],
                         jnp.full((16,), -2.0, jnp.float32), mask=lane0)
      return carry
    lax.fori_loop(0, TOPK, select_k, 0)

    obase = b * (2 * HW)
    lbase = b * (10 * HW)
    for j in range(NSEL // 16):
      sv = sidx_ref[pl.ds(16 * j, 16)]
      gidx_ref[0, pl.ds(16 * j, 16)] = sv + obase
      gidx_ref[1, pl.ds(16 * j, 16)] = sv + (obase + HW)
      gidx_ref[2, pl.ds(16 * j, 16)] = sv + obase
      gidx_ref[3, pl.ds(16 * j, 16)] = sv + (obase + HW)
      for ci in range(10):
        gidx_ref[4 + ci, pl.ds(16 * j, 16)] = sv + (lbase + ci * HW)

    srcs = [off, off, wh, wh] + [lm] * 10
    cps = []
    for ci, src in enumerate(srcs):
      cps.append(pltpu.async_copy(src.at[gidx_ref.at[ci]],
                                  gbuf_ref.at[ci], sem))
    for cp in cps:
      cp.wait()

    for j in range(NSEL // 16):
      ds = pl.ds(16 * j, 16)
      sv_i = sidx_ref[ds]
      ysf = jnp.right_shift(sv_i, 7).astype(jnp.float32)
      xsf = (sv_i & (IMG - 1)).astype(jnp.float32)
      sx = xsf + gbuf_ref[0, ds]
      sy = ysf + gbuf_ref[1, ds]
      hw2 = gbuf_ref[2, ds] * 0.5
      hh2 = gbuf_ref[3, ds] * 0.5
      sc = sval_ref[ds]
      msk = sc > THRESH
      ob_ids[ds] = jnp.where(msk, 0.0, -1.0)
      ob_sc[ds] = jnp.where(msk, sc, -1.0)
      posv = 16 * j + lane
      b4 = posv * 4
      plsc.store_scatter(ob_bb, [b4], jnp.where(msk, sx - hw2, -1.0) * SCALE)
      plsc.store_scatter(ob_bb, [b4 + 1], jnp.where(msk, sy - hh2, -1.0) * SCALE)
      plsc.store_scatter(ob_bb, [b4 + 2], jnp.where(msk, sx + hw2, -1.0) * SCALE)
      plsc.store_scatter(ob_bb, [b4 + 3], jnp.where(msk, sy + hh2, -1.0) * SCALE)
      b10 = posv * 10
      for i5 in range(5):
        lx = gbuf_ref[4 + 2 * i5, ds]
        ly = gbuf_ref[5 + 2 * i5, ds]
        plsc.store_scatter(ob_lm, [b10 + 2 * i5],
                           jnp.where(msk, lx + sx, -1.0) * SCALE)
        plsc.store_scatter(ob_lm, [b10 + 2 * i5 + 1],
                           jnp.where(msk, ly + sy, -1.0) * SCALE)

    pltpu.sync_copy(ob_ids, o_ids.at[pl.ds(b * NSEL, NSEL)])
    pltpu.sync_copy(ob_sc, o_sc.at[pl.ds(b * NSEL, NSEL)])
    pltpu.sync_copy(ob_bb, o_bb.at[pl.ds(b * 4 * NSEL, 4 * NSEL)])
    pltpu.sync_copy(ob_lm, o_lm.at[pl.ds(b * 10 * NSEL, 10 * NSEL)])


@jax.jit
def _run(hm_flat, off_flat, wh_flat, lm_flat):
  mesh = plsc.VectorSubcoreMesh(core_axis_name="c", subcore_axis_name="s")
  f32, i32 = jnp.float32, jnp.int32
  fn = functools.partial(
      pl.kernel,
      mesh=mesh,
      compiler_params=pltpu.CompilerParams(needs_layout_passes=False),
      out_type=(
          jax.ShapeDtypeStruct((BATCH * NSEL,), f32),
          jax.ShapeDtypeStruct((BATCH * NSEL,), f32),
          jax.ShapeDtypeStruct((BATCH * 4 * NSEL,), f32),
          jax.ShapeDtypeStruct((BATCH * 10 * NSEL,), f32),
      ),
      scratch_types=[
          pltpu.VMEM((IMG * PW,), f32),
          pltpu.VMEM((IMG + 2, IMG), f32),
          pltpu.VMEM((HW,), f32),
          pltpu.VMEM((16 * NB,), i32),
          pltpu.VMEM((NB,), i32),
          pltpu.VMEM((NB,), i32),
          pltpu.VMEM((CAP,), f32),
          pltpu.VMEM((CAP,), i32),
          pltpu.VMEM((NSEL,), f32),
          pltpu.VMEM((NSEL,), i32),
          pltpu.VMEM((14, NSEL), i32),
          pltpu.VMEM((14, NSEL), f32),
          pltpu.VMEM((NSEL,), f32),
          pltpu.VMEM((NSEL,), f32),
          pltpu.VMEM((4 * NSEL,), f32),
          pltpu.VMEM((10 * NSEL,), f32),
          pltpu.SemaphoreType.DMA,
      ],
  )(_sc_body)
  return fn(hm_flat, off_flat, wh_flat, lm_flat)


def kernel(heatmap, offset, wh, landmark):
  hm_flat = heatmap.reshape(-1)
  off_flat = offset.reshape(-1)
  wh_flat = wh.reshape(-1)
  lm_flat = landmark.reshape(-1)
  o_ids, o_sc, o_bb, o_lm = _run(hm_flat, off_flat, wh_flat, lm_flat)
  ids = o_ids.reshape(BATCH, NSEL)[:, :TOPK, None]
  scores = o_sc.reshape(BATCH, NSEL)[:, :TOPK, None]
  bboxes = o_bb.reshape(BATCH, NSEL, 4)[:, :TOPK, :]
  landmarks = o_lm.reshape(BATCH, NSEL, 10)[:, :TOPK, :]
  return ids, scores, bboxes, landmarks


# speedup vs baseline: 4.2369x; 1.4947x over previous
import functools

import jax
import jax.numpy as jnp
from jax import lax
from jax.experimental import pallas as pl
from jax.experimental.pallas import tpu as pltpu
from jax.experimental.pallas import tpu_sc as plsc

TOPK = 100
NSEL = 112
NB = 512
CAP = 512
IMG = 128
HW = IMG * IMG
BATCH = 16
SCALE = 4.0
THRESH = 0.01
MOFF = 16


def _sc_body(hm, off, wh, lm, o_ids, o_sc, o_bb, o_lm,
             m_ref, hx_ref, hist_ref, tot_ref, cum_ref, cumv_ref, base_ref,
             cval_ref, cidx_ref, sval_ref, sidx_ref,
             gidx_ref, gbuf_ref, ob_ids, ob_sc, ob_bb, ob_lm, sem):
  wid = lax.axis_index("s") * 2 + lax.axis_index("c")

  @pl.when(wid < BATCH)
  def _():
    b = wid
    lane = lax.iota(jnp.int32, 16)
    neg1f = jnp.full((16,), -1.0, jnp.float32)
    one_i = jnp.full((16,), 1, jnp.int32)
    zero_i = jnp.full((16,), 0, jnp.int32)

    pltpu.sync_copy(hm.at[pl.ds(b * HW, HW)], m_ref.at[pl.ds(MOFF, HW)])

    for j in range(8):
      hx_ref[0, pl.ds(16 * j, 16)] = neg1f
      hx_ref[IMG + 1, pl.ds(16 * j, 16)] = neg1f

    @plsc.parallel_loop(0, IMG, unroll=2)
    def hrow(r):
      base = MOFF + r * IMG
      for j in range(8):
        a = m_ref[pl.ds(base + 16 * j - 1, 16)]
        c0 = m_ref[pl.ds(base + 16 * j, 16)]
        c1 = m_ref[pl.ds(base + 16 * j + 1, 16)]
        if j == 0:
          a = jnp.where(lane == 0, -1.0, a)
        if j == 7:
          c1 = jnp.where(lane == 15, -1.0, c1)
        hx_ref[r + 1, pl.ds(16 * j, 16)] = jnp.maximum(jnp.maximum(a, c0), c1)

    @plsc.parallel_loop(0, 16 * NB // 16, unroll=4)
    def zh(i):
      hist_ref[pl.ds(i * 16, 16)] = zero_i

    @plsc.parallel_loop(0, IMG, unroll=2)
    def prow(r):
      for j in range(8):
        ctr = m_ref[pl.ds(MOFF + r * IMG + 16 * j, 16)]
        v0 = hx_ref[r, pl.ds(16 * j, 16)]
        v1 = hx_ref[r + 1, pl.ds(16 * j, 16)]
        v2 = hx_ref[r + 2, pl.ds(16 * j, 16)]
        pool = jnp.maximum(jnp.maximum(v0, v1), v2)
        keep = ctr >= pool
        m_ref[pl.ds(MOFF + r * IMG + 16 * j, 16)] = jnp.where(keep, ctr, -1.0)
        bidx = jnp.minimum((ctr * float(NB)).astype(jnp.int32), NB - 1)
        plsc.addupdate_scatter(hist_ref, [lane * NB + bidx], one_i, mask=keep)

    @plsc.parallel_loop(0, NB // 16, unroll=2)
    def tchunk(j):
      acc = hist_ref[pl.ds(j * 16, 16)]
      for l in range(1, 16):
        acc = acc + hist_ref[pl.ds(l * NB + j * 16, 16)]
      tot_ref[pl.ds(j * 16, 16)] = acc

    def cchunk(j, carry):
      cs = jnp.cumsum(tot_ref[pl.ds(16 * j, 16)]) + carry
      cum_ref[pl.ds(16 * j, 16)] = cs
      return jnp.max(cs)
    tot_peaks = lax.fori_loop(0, NB // 16, cchunk, jnp.int32(0))

    def kchunk(j, kacc):
      tch = tot_ref[pl.ds(16 * j, 16)]
      cs = cum_ref[pl.ds(16 * j, 16)]
      sfx = tot_peaks - cs + tch
      return kacc + jnp.sum((sfx >= TOPK).astype(jnp.int32))
    t = lax.fori_loop(0, NB // 16, kchunk, jnp.int32(0)) - 1

    @plsc.parallel_loop(0, CAP // 16, unroll=4)
    def pfc(i):
      cval_ref[pl.ds(i * 16, 16)] = neg1f
    for j in range(NSEL // 16):
      sval_ref[pl.ds(16 * j, 16)] = neg1f
      sidx_ref[pl.ds(16 * j, 16)] = zero_i

    @plsc.parallel_loop(0, HW // 16, unroll=2)
    def compa(i):
      mv = m_ref[pl.ds(MOFF + 16 * i, 16)]
      keep = mv >= 0.0
      bidx = jnp.minimum((mv * float(NB)).astype(jnp.int32), NB - 1)
      sel = keep & (bidx >= t)
      cumv_ref[pl.ds(16 * i, 16)] = jnp.cumsum(sel.astype(jnp.int32))

    def compb(p, carry):
      tots = plsc.load_gather(cumv_ref, [256 * p + 16 * lane + 15])
      incl = jnp.cumsum(tots)
      base_ref[pl.ds(16 * p, 16)] = incl - tots + carry
      return carry + jnp.max(incl)
    ncand = lax.fori_loop(0, HW // 256, compb, jnp.int32(0))
    nv = (jnp.minimum(ncand, CAP) + 15) // 16

    @plsc.parallel_loop(0, HW // 16, unroll=2)
    def compc(i):
      mv = m_ref[pl.ds(MOFF + 16 * i, 16)]
      keep = mv >= 0.0
      bidx = jnp.minimum((mv * float(NB)).astype(jnp.int32), NB - 1)
      sel = keep & (bidx >= t)
      cs = cumv_ref[pl.ds(16 * i, 16)]
      bvec = plsc.load_gather(base_ref, [zero_i + i])
      dest = bvec + cs - 1
      sel = sel & (dest < CAP)
      plsc.store_scatter(cval_ref, [dest], mv, mask=sel)
      plsc.store_scatter(cidx_ref, [dest], 16 * i + lane, mask=sel)

    def select_k(k, carry):
      def scan_j(j, bc):
        bv, bp = bc
        v = cval_ref[pl.ds(16 * j, 16)]
        p = 16 * j + lane
        upd = v > bv
        return (jnp.where(upd, v, bv), jnp.where(upd, p, bp))
      bv, bp = lax.fori_loop(0, nv, scan_j,
                             (jnp.full((16,), -2.0, jnp.float32), zero_i))
      mx = jnp.max(bv)
      pmin = jnp.min(jnp.where(bv == mx, bp, CAP))
      pminv = zero_i + pmin
      kv = zero_i + k
      lane0 = lane == 0
      siv = plsc.load_gather(cidx_ref, [pminv])
      plsc.store_scatter(sval_ref, [kv], jnp.zeros((16,), jnp.float32) + mx,
                         mask=lane0)
      plsc.store_scatter(sidx_ref, [kv], siv, mask=lane0)
      plsc.store_scatter(cval_ref, [pminv],
                         jnp.full((16,), -2.0, jnp.float32), mask=lane0)
      return carry
    lax.fori_loop(0, TOPK, select_k, 0)

    obase = b * (2 * HW)
    lbase = b * (10 * HW)
    for j in range(NSEL // 16):
      sv = sidx_ref[pl.ds(16 * j, 16)]
      gidx_ref[0, pl.ds(16 * j, 16)] = sv + obase
      gidx_ref[1, pl.ds(16 * j, 16)] = sv + (obase + HW)
      gidx_ref[2, pl.ds(16 * j, 16)] = sv + obase
      gidx_ref[3, pl.ds(16 * j, 16)] = sv + (obase + HW)
      for ci in range(10):
        gidx_ref[4 + ci, pl.ds(16 * j, 16)] = sv + (lbase + ci * HW)

    srcs = [off, off, wh, wh] + [lm] * 10
    cps = []
    for ci, src in enumerate(srcs):
      cps.append(pltpu.async_copy(src.at[gidx_ref.at[ci]],
                                  gbuf_ref.at[ci], sem))
    for cp in cps:
      cp.wait()

    for j in range(NSEL // 16):
      ds = pl.ds(16 * j, 16)
      sv_i = sidx_ref[ds]
      ysf = jnp.right_shift(sv_i, 7).astype(jnp.float32)
      xsf = (sv_i & (IMG - 1)).astype(jnp.float32)
      sx = xsf + gbuf_ref[0, ds]
      sy = ysf + gbuf_ref[1, ds]
      hw2 = gbuf_ref[2, ds] * 0.5
      hh2 = gbuf_ref[3, ds] * 0.5
      sc = sval_ref[ds]
      msk = sc > THRESH
      ob_ids[ds] = jnp.where(msk, 0.0, -1.0)
      ob_sc[ds] = jnp.where(msk, sc, -1.0)
      posv = 16 * j + lane
      b4 = posv * 4
      plsc.store_scatter(ob_bb, [b4], jnp.where(msk, sx - hw2, -1.0) * SCALE)
      plsc.store_scatter(ob_bb, [b4 + 1], jnp.where(msk, sy - hh2, -1.0) * SCALE)
      plsc.store_scatter(ob_bb, [b4 + 2], jnp.where(msk, sx + hw2, -1.0) * SCALE)
      plsc.store_scatter(ob_bb, [b4 + 3], jnp.where(msk, sy + hh2, -1.0) * SCALE)
      b10 = posv * 10
      for i5 in range(5):
        lx = gbuf_ref[4 + 2 * i5, ds]
        ly = gbuf_ref[5 + 2 * i5, ds]
        plsc.store_scatter(ob_lm, [b10 + 2 * i5],
                           jnp.where(msk, lx + sx, -1.0) * SCALE)
        plsc.store_scatter(ob_lm, [b10 + 2 * i5 + 1],
                           jnp.where(msk, ly + sy, -1.0) * SCALE)

    pltpu.sync_copy(ob_ids, o_ids.at[pl.ds(b * NSEL, NSEL)])
    pltpu.sync_copy(ob_sc, o_sc.at[pl.ds(b * NSEL, NSEL)])
    pltpu.sync_copy(ob_bb, o_bb.at[pl.ds(b * 4 * NSEL, 4 * NSEL)])
    pltpu.sync_copy(ob_lm, o_lm.at[pl.ds(b * 10 * NSEL, 10 * NSEL)])


@jax.jit
def _run(hm_flat, off_flat, wh_flat, lm_flat):
  mesh = plsc.VectorSubcoreMesh(core_axis_name="c", subcore_axis_name="s")
  f32, i32 = jnp.float32, jnp.int32
  fn = functools.partial(
      pl.kernel,
      mesh=mesh,
      compiler_params=pltpu.CompilerParams(needs_layout_passes=False),
      out_type=(
          jax.ShapeDtypeStruct((BATCH * NSEL,), f32),
          jax.ShapeDtypeStruct((BATCH * NSEL,), f32),
          jax.ShapeDtypeStruct((BATCH * 4 * NSEL,), f32),
          jax.ShapeDtypeStruct((BATCH * 10 * NSEL,), f32),
      ),
      scratch_types=[
          pltpu.VMEM((MOFF + HW + MOFF,), f32),
          pltpu.VMEM((IMG + 2, IMG), f32),
          pltpu.VMEM((16 * NB,), i32),
          pltpu.VMEM((NB,), i32),
          pltpu.VMEM((NB,), i32),
          pltpu.VMEM((HW,), i32),
          pltpu.VMEM((HW // 16,), i32),
          pltpu.VMEM((CAP,), f32),
          pltpu.VMEM((CAP,), i32),
          pltpu.VMEM((NSEL,), f32),
          pltpu.VMEM((NSEL,), i32),
          pltpu.VMEM((14, NSEL), i32),
          pltpu.VMEM((14, NSEL), f32),
          pltpu.VMEM((NSEL,), f32),
          pltpu.VMEM((NSEL,), f32),
          pltpu.VMEM((4 * NSEL,), f32),
          pltpu.VMEM((10 * NSEL,), f32),
          pltpu.SemaphoreType.DMA,
      ],
  )(_sc_body)
  return fn(hm_flat, off_flat, wh_flat, lm_flat)


def kernel(heatmap, offset, wh, landmark):
  hm_flat = heatmap.reshape(-1)
  off_flat = offset.reshape(-1)
  wh_flat = wh.reshape(-1)
  lm_flat = landmark.reshape(-1)
  o_ids, o_sc, o_bb, o_lm = _run(hm_flat, off_flat, wh_flat, lm_flat)
  ids = o_ids.reshape(BATCH, NSEL)[:, :TOPK, None]
  scores = o_sc.reshape(BATCH, NSEL)[:, :TOPK, None]
  bboxes = o_bb.reshape(BATCH, NSEL, 4)[:, :TOPK, :]
  landmarks = o_lm.reshape(BATCH, NSEL, 10)[:, :TOPK, :]
  return ids, scores, bboxes, landmarks


# speedup vs baseline: 4.6387x; 1.0948x over previous
import functools

import jax
import jax.numpy as jnp
from jax import lax
from jax.experimental import pallas as pl
from jax.experimental.pallas import tpu as pltpu
from jax.experimental.pallas import tpu_sc as plsc

TOPK = 100
NSEL = 112
NB = 512
CAP = 512
IMG = 128
HW = IMG * IMG
HHW = HW // 2
BATCH = 16
SCALE = 4.0
THRESH = 0.01
MOFF = 16
LROWS = 65


def _sc_body(hm, off, wh, lm, o_ids, o_sc, o_bb, o_lm,
             m_ref, hx_ref, hist_ref, tot_ref, ptot_ref, cum_ref,
             cumv_ref, base_ref, cval_ref, cidx_ref, pcval_ref, pcidx_ref,
             sval_ref, sidx_ref, gidx_ref, gbuf_ref,
             ob_ids, ob_sc, ob_bb, ob_lm,
             sh_tot, sh_cval, sh_cidx, sem):
  c = lax.axis_index("c")
  s = lax.axis_index("s")
  b = c * 8 + jnp.right_shift(s, 1)
  h = s & 1
  lane = lax.iota(jnp.int32, 16)
  neg1f = jnp.full((16,), -1.0, jnp.float32)
  one_i = jnp.full((16,), 1, jnp.int32)
  zero_i = jnp.full((16,), 0, jnp.int32)

  pltpu.sync_copy(hm.at[pl.ds(b * HW + h * 63 * IMG, LROWS * IMG)],
                  m_ref.at[pl.ds(MOFF, LROWS * IMG)])

  for j in range(8):
    hx_ref[0, pl.ds(16 * j, 16)] = neg1f
    hx_ref[LROWS + 1, pl.ds(16 * j, 16)] = neg1f

  @plsc.parallel_loop(0, LROWS, unroll=2)
  def hrow(r):
    base = MOFF + r * IMG
    for j in range(8):
      a = m_ref[pl.ds(base + 16 * j - 1, 16)]
      c0 = m_ref[pl.ds(base + 16 * j, 16)]
      c1 = m_ref[pl.ds(base + 16 * j + 1, 16)]
      if j == 0:
        a = jnp.where(lane == 0, -1.0, a)
      if j == 7:
        c1 = jnp.where(lane == 15, -1.0, c1)
      hx_ref[r + 1, pl.ds(16 * j, 16)] = jnp.maximum(jnp.maximum(a, c0), c1)

  @plsc.parallel_loop(0, 16 * NB // 16, unroll=4)
  def zh(i):
    hist_ref[pl.ds(i * 16, 16)] = zero_i

  @plsc.parallel_loop(0, IMG // 2, unroll=2)
  def prow(i):
    li = i + h
    for j in range(8):
      ctr = m_ref[pl.ds(MOFF + li * IMG + 16 * j, 16)]
      v0 = hx_ref[li, pl.ds(16 * j, 16)]
      v1 = hx_ref[li + 1, pl.ds(16 * j, 16)]
      v2 = hx_ref[li + 2, pl.ds(16 * j, 16)]
      pool = jnp.maximum(jnp.maximum(v0, v1), v2)
      keep = ctr >= pool
      m_ref[pl.ds(MOFF + li * IMG + 16 * j, 16)] = jnp.where(keep, ctr, -1.0)
      bidx = jnp.minimum((ctr * float(NB)).astype(jnp.int32), NB - 1)
      plsc.addupdate_scatter(hist_ref, [lane * NB + bidx], one_i, mask=keep)

  @plsc.parallel_loop(0, NB // 16, unroll=2)
  def tchunk(j):
    acc = hist_ref[pl.ds(j * 16, 16)]
    for l in range(1, 16):
      acc = acc + hist_ref[pl.ds(l * NB + j * 16, 16)]
    tot_ref[pl.ds(j * 16, 16)] = acc

  partner = s + 1 - 2 * h
  pltpu.sync_copy(tot_ref, sh_tot.at[s])
  plsc.subcore_barrier()
  pltpu.sync_copy(sh_tot.at[partner], ptot_ref)
  @plsc.parallel_loop(0, NB // 16, unroll=2)
  def tmerge(j):
    tot_ref[pl.ds(16 * j, 16)] = (tot_ref[pl.ds(16 * j, 16)]
                                  + ptot_ref[pl.ds(16 * j, 16)])

  def cchunk(j, carry):
    cs = jnp.cumsum(tot_ref[pl.ds(16 * j, 16)]) + carry
    cum_ref[pl.ds(16 * j, 16)] = cs
    return jnp.max(cs)
  tot_peaks = lax.fori_loop(0, NB // 16, cchunk, jnp.int32(0))

  def kchunk(j, kacc):
    tch = tot_ref[pl.ds(16 * j, 16)]
    cs = cum_ref[pl.ds(16 * j, 16)]
    sfx = tot_peaks - cs + tch
    return kacc + jnp.sum((sfx >= TOPK).astype(jnp.int32))
  t = lax.fori_loop(0, NB // 16, kchunk, jnp.int32(0)) - 1

  @plsc.parallel_loop(0, 2 * CAP // 16, unroll=4)
  def pfc(i):
    cval_ref[pl.ds(i * 16, 16)] = neg1f
  for j in range(NSEL // 16):
    sval_ref[pl.ds(16 * j, 16)] = neg1f
    sidx_ref[pl.ds(16 * j, 16)] = zero_i

  @plsc.parallel_loop(0, HHW // 16, unroll=2)
  def compa(i):
    mv = m_ref[pl.ds(MOFF + IMG * h + 16 * i, 16)]
    keep = mv >= 0.0
    bidx = jnp.minimum((mv * float(NB)).astype(jnp.int32), NB - 1)
    sel = keep & (bidx >= t)
    cumv_ref[pl.ds(16 * i, 16)] = jnp.cumsum(sel.astype(jnp.int32))

  def compb(p, carry):
    tots = plsc.load_gather(cumv_ref, [256 * p + 16 * lane + 15])
    incl = jnp.cumsum(tots)
    base_ref[pl.ds(16 * p, 16)] = incl - tots + carry
    return carry + jnp.max(incl)
  ncand = lax.fori_loop(0, HHW // 256, compb, jnp.int32(0))
  nc0 = jnp.minimum(ncand, CAP)

  @plsc.parallel_loop(0, HHW // 16, unroll=2)
  def compc(i):
    mv = m_ref[pl.ds(MOFF + IMG * h + 16 * i, 16)]
    keep = mv >= 0.0
    bidx = jnp.minimum((mv * float(NB)).astype(jnp.int32), NB - 1)
    sel = keep & (bidx >= t)
    cs = cumv_ref[pl.ds(16 * i, 16)]
    bvec = plsc.load_gather(base_ref, [zero_i + i])
    dest = bvec + cs - 1
    sel = sel & (dest < CAP)
    plsc.store_scatter(cval_ref, [dest], mv, mask=sel)
    plsc.store_scatter(cidx_ref, [dest], HHW * h + 16 * i + lane, mask=sel)

  pltpu.sync_copy(cval_ref.at[pl.ds(0, CAP)], sh_cval.at[s])
  pltpu.sync_copy(cidx_ref.at[pl.ds(0, CAP)], sh_cidx.at[s])
  plsc.subcore_barrier()

  @pl.when(h == 0)
  def _tail():
    pltpu.sync_copy(sh_cval.at[s + 1], pcval_ref)
    pltpu.sync_copy(sh_cidx.at[s + 1], pcidx_ref)
    cacc = zero_i
    for u in range(CAP // 16):
      pv = pcval_ref[pl.ds(16 * u, 16)]
      cval_ref[pl.ds(nc0 + 16 * u, 16)] = pv
      cidx_ref[pl.ds(nc0 + 16 * u, 16)] = pcidx_ref[pl.ds(16 * u, 16)]
      cacc = cacc + (pv >= 0.0).astype(jnp.int32)
    pc = jnp.sum(cacc)
    nv = (jnp.minimum(nc0 + pc, 2 * CAP) + 15) // 16

    def select_k(k, carry):
      def scan_j(j, bc):
        bv, bp = bc
        v = cval_ref[pl.ds(16 * j, 16)]
        p = 16 * j + lane
        upd = v > bv
        return (jnp.where(upd, v, bv), jnp.where(upd, p, bp))
      bv, bp = lax.fori_loop(0, nv, scan_j,
                             (jnp.full((16,), -2.0, jnp.float32), zero_i))
      mx = jnp.max(bv)
      pmin = jnp.min(jnp.where(bv == mx, bp, 2 * CAP))
      pminv = zero_i + pmin
      kv = zero_i + k
      lane0 = lane == 0
      siv = plsc.load_gather(cidx_ref, [pminv])
      plsc.store_scatter(sval_ref, [kv], jnp.zeros((16,), jnp.float32) + mx,
                         mask=lane0)
      plsc.store_scatter(sidx_ref, [kv], siv, mask=lane0)
      plsc.store_scatter(cval_ref, [pminv],
                         jnp.full((16,), -2.0, jnp.float32), mask=lane0)
      return carry
    lax.fori_loop(0, TOPK, select_k, 0)

    obase = b * (2 * HW)
    lbase = b * (10 * HW)
    for j in range(NSEL // 16):
      sv = sidx_ref[pl.ds(16 * j, 16)]
      gidx_ref[0, pl.ds(16 * j, 16)] = sv + obase
      gidx_ref[1, pl.ds(16 * j, 16)] = sv + (obase + HW)
      gidx_ref[2, pl.ds(16 * j, 16)] = sv + obase
      gidx_ref[3, pl.ds(16 * j, 16)] = sv + (obase + HW)
      for ci in range(10):
        gidx_ref[4 + ci, pl.ds(16 * j, 16)] = sv + (lbase + ci * HW)

    srcs = [off, off, wh, wh] + [lm] * 10
    cps = []
    for ci, src in enumerate(srcs):
      cps.append(pltpu.async_copy(src.at[gidx_ref.at[ci]],
                                  gbuf_ref.at[ci], sem))
    for cp in cps:
      cp.wait()

    for j in range(NSEL // 16):
      ds = pl.ds(16 * j, 16)
      sv_i = sidx_ref[ds]
      ysf = jnp.right_shift(sv_i, 7).astype(jnp.float32)
      xsf = (sv_i & (IMG - 1)).astype(jnp.float32)
      sx = xsf + gbuf_ref[0, ds]
      sy = ysf + gbuf_ref[1, ds]
      hw2 = gbuf_ref[2, ds] * 0.5
      hh2 = gbuf_ref[3, ds] * 0.5
      sc = sval_ref[ds]
      msk = sc > THRESH
      ob_ids[ds] = jnp.where(msk, 0.0, -1.0)
      ob_sc[ds] = jnp.where(msk, sc, -1.0)
      posv = 16 * j + lane
      b4 = posv * 4
      plsc.store_scatter(ob_bb, [b4], jnp.where(msk, sx - hw2, -1.0) * SCALE)
      plsc.store_scatter(ob_bb, [b4 + 1],
                         jnp.where(msk, sy - hh2, -1.0) * SCALE)
      plsc.store_scatter(ob_bb, [b4 + 2],
                         jnp.where(msk, sx + hw2, -1.0) * SCALE)
      plsc.store_scatter(ob_bb, [b4 + 3],
                         jnp.where(msk, sy + hh2, -1.0) * SCALE)
      b10 = posv * 10
      for i5 in range(5):
        lx = gbuf_ref[4 + 2 * i5, ds]
        ly = gbuf_ref[5 + 2 * i5, ds]
        plsc.store_scatter(ob_lm, [b10 + 2 * i5],
                           jnp.where(msk, lx + sx, -1.0) * SCALE)
        plsc.store_scatter(ob_lm, [b10 + 2 * i5 + 1],
                           jnp.where(msk, ly + sy, -1.0) * SCALE)

    pltpu.sync_copy(ob_ids, o_ids.at[pl.ds(b * NSEL, NSEL)])
    pltpu.sync_copy(ob_sc, o_sc.at[pl.ds(b * NSEL, NSEL)])
    pltpu.sync_copy(ob_bb, o_bb.at[pl.ds(b * 4 * NSEL, 4 * NSEL)])
    pltpu.sync_copy(ob_lm, o_lm.at[pl.ds(b * 10 * NSEL, 10 * NSEL)])


@jax.jit
def _run(hm_flat, off_flat, wh_flat, lm_flat):
  mesh = plsc.VectorSubcoreMesh(core_axis_name="c", subcore_axis_name="s")
  f32, i32 = jnp.float32, jnp.int32
  fn = functools.partial(
      pl.kernel,
      mesh=mesh,
      compiler_params=pltpu.CompilerParams(needs_layout_passes=False),
      out_type=(
          jax.ShapeDtypeStruct((BATCH * NSEL,), f32),
          jax.ShapeDtypeStruct((BATCH * NSEL,), f32),
          jax.ShapeDtypeStruct((BATCH * 4 * NSEL,), f32),
          jax.ShapeDtypeStruct((BATCH * 10 * NSEL,), f32),
      ),
      scratch_types=[
          pltpu.VMEM((MOFF + LROWS * IMG + MOFF,), f32),
          pltpu.VMEM((LROWS + 2, IMG), f32),
          pltpu.VMEM((16 * NB,), i32),
          pltpu.VMEM((NB,), i32),
          pltpu.VMEM((NB,), i32),
          pltpu.VMEM((NB,), i32),
          pltpu.VMEM((HHW,), i32),
          pltpu.VMEM((HHW // 16,), i32),
          pltpu.VMEM((2 * CAP,), f32),
          pltpu.VMEM((2 * CAP,), i32),
          pltpu.VMEM((CAP,), f32),
          pltpu.VMEM((CAP,), i32),
          pltpu.VMEM((NSEL,), f32),
          pltpu.VMEM((NSEL,), i32),
          pltpu.VMEM((14, NSEL), i32),
          pltpu.VMEM((14, NSEL), f32),
          pltpu.VMEM((NSEL,), f32),
          pltpu.VMEM((NSEL,), f32),
          pltpu.VMEM((4 * NSEL,), f32),
          pltpu.VMEM((10 * NSEL,), f32),
          pltpu.VMEM_SHARED((16, NB), i32),
          pltpu.VMEM_SHARED((16, CAP), f32),
          pltpu.VMEM_SHARED((16, CAP), i32),
          pltpu.SemaphoreType.DMA,
      ],
  )(_sc_body)
  return fn(hm_flat, off_flat, wh_flat, lm_flat)


def kernel(heatmap, offset, wh, landmark):
  hm_flat = heatmap.reshape(-1)
  off_flat = offset.reshape(-1)
  wh_flat = wh.reshape(-1)
  lm_flat = landmark.reshape(-1)
  o_ids, o_sc, o_bb, o_lm = _run(hm_flat, off_flat, wh_flat, lm_flat)
  ids = o_ids.reshape(BATCH, NSEL)[:, :TOPK, None]
  scores = o_sc.reshape(BATCH, NSEL)[:, :TOPK, None]
  bboxes = o_bb.reshape(BATCH, NSEL, 4)[:, :TOPK, :]
  landmarks = o_lm.reshape(BATCH, NSEL, 10)[:, :TOPK, :]
  return ids, scores, bboxes, landmarks
